# baseline (speedup 1.0000x reference)
import functools

import jax
import jax.numpy as jnp
from jax.experimental import pallas as pl

_BLKC = 1000


def _body(lab_ref, out_ref):
    cb = pl.program_id(1)
    cls = jax.lax.broadcasted_iota(jnp.int32, (_BLKC, 1), 0) + cb * _BLKC
    out_ref[0] = (lab_ref[0] == cls).astype(jnp.float32)


def kernel(label, N):
    n_cls = 1000
    b, l = label.shape
    lab_t = label.T.reshape(l, 1, b)
    out = pl.pallas_call(
        _body,
        grid=(l, n_cls // _BLKC),
        in_specs=[pl.BlockSpec((1, 1, b), lambda j, cb: (j, 0, 0))],
        out_specs=pl.BlockSpec((1, _BLKC, b), lambda j, cb: (j, cb, 0)),
        out_shape=jax.ShapeDtypeStruct((l, n_cls, b), jnp.float32),
    )(lab_t)
    return out.transpose(1, 2, 0)
